# baseline (speedup 1.0000x reference)
import functools

import jax
import jax.numpy as jnp
from jax import lax
from jax.experimental import pallas as pl
from jax.experimental.pallas import tpu as pltpu
from jax.experimental.pallas import tpu_sc as plsc

N = 64 * 8192
NC, NS, L = 2, 16, 16
NW = NC * NS
CHUNK = N // NW
NBLK = CHUNK // L

_sc_mesh = plsc.VectorSubcoreMesh(core_axis_name="c", subcore_axis_name="s")


@functools.partial(
    pl.kernel,
    out_type=[
        jax.ShapeDtypeStruct((NW, L), jnp.float32),
        jax.ShapeDtypeStruct((NW, L), jnp.int32),
    ],
    mesh=_sc_mesh,
    scratch_types=[
        pltpu.VMEM((CHUNK,), jnp.float32),
        pltpu.VMEM((L,), jnp.float32),
        pltpu.VMEM((L,), jnp.int32),
    ],
)
def _sc_local_argmin(a_hbm, vals_out, idxs_out, buf, vmin_ref, vidx_ref):
    c = lax.axis_index("c")
    s = lax.axis_index("s")
    wid = s * NC + c
    base = wid * CHUNK

    pltpu.sync_copy(a_hbm.at[pl.ds(base, CHUNK)], buf)

    lane = lax.iota(jnp.int32, L)
    vmin0 = buf[pl.ds(0, L)]
    vidx0 = base + lane

    def body(i, carry):
        vmin, vidx = carry
        v = buf[pl.ds(i * L, L)]
        idx = (base + i * L) + lane
        upd = v < vmin
        return jnp.where(upd, v, vmin), jnp.where(upd, idx, vidx)

    vmin, vidx = lax.fori_loop(1, NBLK, body, (vmin0, vidx0))

    vmin_ref[...] = vmin
    vidx_ref[...] = vidx
    pltpu.sync_copy(vmin_ref, vals_out.at[wid])
    pltpu.sync_copy(vidx_ref, idxs_out.at[wid])


def _merge_body(vals_ref, idxs_ref, out_ref):
    vals = vals_ref[...]
    idxs = idxs_ref[...]
    m = jnp.min(vals)
    cand = jnp.where(vals == m, idxs, jnp.int32(2**31 - 1))
    out_ref[0, 0] = jnp.min(cand)


_merge = pl.pallas_call(
    _merge_body,
    out_shape=jax.ShapeDtypeStruct((1, 1), jnp.int32),
    out_specs=pl.BlockSpec(memory_space=pltpu.SMEM),
)


def kernel(a):
    vals, idxs = _sc_local_argmin(a.reshape(-1))
    out = _merge(vals, idxs)
    return out[0, 0].astype(jnp.int64)


# speedup vs baseline: 1.1978x; 1.1978x over previous
import functools

import jax
import jax.numpy as jnp
from jax import lax
from jax.experimental import pallas as pl
from jax.experimental.pallas import tpu as pltpu
from jax.experimental.pallas import tpu_sc as plsc

R, C = 64, 8192
NC, NS, L = 2, 16, 16
NW = NC * NS
RPW = R // NW
CHUNKS = 4
CHUNK = RPW * C // CHUNKS
NBLK = CHUNK // L
U = 4

_sc_mesh = plsc.VectorSubcoreMesh(core_axis_name="c", subcore_axis_name="s")


@functools.partial(
    pl.kernel,
    out_type=[
        jax.ShapeDtypeStruct((NW, L), jnp.float32),
        jax.ShapeDtypeStruct((NW, L), jnp.int32),
    ],
    mesh=_sc_mesh,
    scratch_types=[
        pltpu.VMEM((RPW * C,), jnp.float32),
        pltpu.VMEM((L,), jnp.float32),
        pltpu.VMEM((L,), jnp.int32),
        [pltpu.SemaphoreType.DMA] * CHUNKS,
    ],
)
def _sc_local_argmin(a_hbm, vals_out, idxs_out, buf, vmin_ref, vidx_ref, sems):
    c = lax.axis_index("c")
    s = lax.axis_index("s")
    wid = s * NC + c
    row0 = wid * RPW
    base = row0 * C

    copies = []
    for k in range(CHUNKS):
        r = k // (CHUNKS // RPW)
        col = (k % (CHUNKS // RPW)) * CHUNK
        copies.append(
            pltpu.async_copy(
                a_hbm.at[row0 + r, pl.ds(col, CHUNK)],
                buf.at[pl.ds(k * CHUNK, CHUNK)],
                sems[k],
            )
        )

    lane = lax.iota(jnp.int32, L)
    big = jnp.float32(jnp.inf)
    vmins = [jnp.full((L,), big, jnp.float32) for _ in range(U)]
    vidxs = [jnp.zeros((L,), jnp.int32) for _ in range(U)]

    for k in range(CHUNKS):
        copies[k].wait()
        cbase = k * CHUNK

        init = tuple(vmins) + tuple(
            base + cbase + u * L + lane for u in range(U)
        ) + tuple(vidxs)

        @plsc.parallel_loop(0, NBLK // U, carry=init, unroll=2)
        def body(i, carry):
            vm = list(carry[:U])
            cur = list(carry[U : 2 * U])
            vi = list(carry[2 * U :])
            for u in range(U):
                v = buf[pl.ds(cbase + (i * U + u) * L, L)]
                upd = v < vm[u]
                vm[u] = jnp.where(upd, v, vm[u])
                vi[u] = jnp.where(upd, cur[u], vi[u])
                cur[u] = cur[u] + U * L
            return tuple(vm) + tuple(cur) + tuple(vi)

        out = body
        vmins = list(out[:U])
        vidxs = list(out[2 * U :])

    vmin, vidx = vmins[0], vidxs[0]
    for u in range(1, U):
        upd = (vmins[u] < vmin) | ((vmins[u] == vmin) & (vidxs[u] < vidx))
        vmin = jnp.where(upd, vmins[u], vmin)
        vidx = jnp.where(upd, vidxs[u], vidx)

    vmin_ref[...] = vmin
    vidx_ref[...] = vidx
    pltpu.sync_copy(vmin_ref, vals_out.at[wid])
    pltpu.sync_copy(vidx_ref, idxs_out.at[wid])


def _merge_body(vals_ref, idxs_ref, out_ref):
    vals = vals_ref[...]
    idxs = idxs_ref[...]
    m = jnp.min(vals)
    cand = jnp.where(vals == m, idxs, jnp.int32(2**31 - 1))
    out_ref[0, 0] = jnp.min(cand)


_merge = pl.pallas_call(
    _merge_body,
    out_shape=jax.ShapeDtypeStruct((1, 1), jnp.int32),
    out_specs=pl.BlockSpec(memory_space=pltpu.SMEM),
)


def kernel(a):
    vals, idxs = _sc_local_argmin(a)
    out = _merge(vals, idxs)
    return out[0, 0].astype(jnp.int64)


# speedup vs baseline: 1.2243x; 1.0222x over previous
import functools

import jax
import jax.numpy as jnp
from jax import lax
from jax.experimental import pallas as pl
from jax.experimental.pallas import tpu as pltpu
from jax.experimental.pallas import tpu_sc as plsc

R, C = 64, 8192
NS, L = 16, 16
RPW = R // NS
CHUNKS = 4
CHUNK = RPW * C // CHUNKS
NBLK = CHUNK // L
U = 4

_sc_mesh = plsc.VectorSubcoreMesh(
    core_axis_name="c", subcore_axis_name="s", num_cores=1
)


@functools.partial(
    pl.kernel,
    out_type=jax.ShapeDtypeStruct((L,), jnp.int32),
    mesh=_sc_mesh,
    scratch_types=[
        pltpu.VMEM((RPW * C,), jnp.float32),
        pltpu.VMEM((L,), jnp.float32),
        pltpu.VMEM((L,), jnp.int32),
        pltpu.VMEM((NS * L,), jnp.float32),
        pltpu.VMEM((NS * L,), jnp.int32),
        pltpu.VMEM_SHARED((NS * L,), jnp.float32),
        pltpu.VMEM_SHARED((NS * L,), jnp.int32),
        [pltpu.SemaphoreType.DMA] * CHUNKS,
    ],
)
def _sc_argmin(
    a_hbm, out, buf, vmin_ref, vidx_ref, mv_ref, mi_ref, sh_v, sh_i, sems
):
    s = lax.axis_index("s")
    row0 = s * RPW
    base = row0 * C

    copies = []
    for k in range(CHUNKS):
        copies.append(
            pltpu.async_copy(
                a_hbm.at[row0 + k],
                buf.at[pl.ds(k * CHUNK, CHUNK)],
                sems[k],
            )
        )

    lane = lax.iota(jnp.int32, L)
    big = jnp.float32(jnp.inf)
    vmins = [jnp.full((L,), big, jnp.float32) for _ in range(U)]
    vidxs = [jnp.zeros((L,), jnp.int32) for _ in range(U)]

    for k in range(CHUNKS):
        copies[k].wait()
        cbase = k * CHUNK

        init = tuple(vmins) + tuple(
            base + cbase + u * L + lane for u in range(U)
        ) + tuple(vidxs)

        @plsc.parallel_loop(0, NBLK // U, carry=init, unroll=2)
        def body(i, carry):
            vm = list(carry[:U])
            cur = list(carry[U : 2 * U])
            vi = list(carry[2 * U :])
            for u in range(U):
                v = buf[pl.ds(cbase + (i * U + u) * L, L)]
                upd = v < vm[u]
                vm[u] = jnp.where(upd, v, vm[u])
                vi[u] = jnp.where(upd, cur[u], vi[u])
                cur[u] = cur[u] + U * L
            return tuple(vm) + tuple(cur) + tuple(vi)

        out_carry = body
        vmins = list(out_carry[:U])
        vidxs = list(out_carry[2 * U :])

    vmin, vidx = vmins[0], vidxs[0]
    for u in range(1, U):
        upd = (vmins[u] < vmin) | ((vmins[u] == vmin) & (vidxs[u] < vidx))
        vmin = jnp.where(upd, vmins[u], vmin)
        vidx = jnp.where(upd, vidxs[u], vidx)

    vmin_ref[...] = vmin
    vidx_ref[...] = vidx
    pltpu.sync_copy(vmin_ref, sh_v.at[pl.ds(s * L, L)])
    pltpu.sync_copy(vidx_ref, sh_i.at[pl.ds(s * L, L)])
    plsc.subcore_barrier()

    @pl.when(s == 0)
    def _final_merge():
        pltpu.sync_copy(sh_v, mv_ref)
        pltpu.sync_copy(sh_i, mi_ref)
        fv = mv_ref[pl.ds(0, L)]
        fi = mi_ref[pl.ds(0, L)]
        for w in range(1, NS):
            wv = mv_ref[pl.ds(w * L, L)]
            wi = mi_ref[pl.ds(w * L, L)]
            upd = (wv < fv) | ((wv == fv) & (wi < fi))
            fv = jnp.where(upd, wv, fv)
            fi = jnp.where(upd, wi, fi)
        bv = fv[0]
        bi = fi[0]
        for l in range(1, L):
            v = fv[l]
            i = fi[l]
            upd = (v < bv) | ((v == bv) & (i < bi))
            bv = jnp.where(upd, v, bv)
            bi = jnp.where(upd, i, bi)
        vidx_ref[...] = jnp.full((L,), bi, jnp.int32)
        pltpu.sync_copy(vidx_ref, out)


def kernel(a):
    idx = _sc_argmin(a)
    return idx[0].astype(jnp.int64)
